# Initial kernel scaffold; baseline (speedup 1.0000x reference)
import jax
import jax.numpy as jnp
from jax.experimental import pallas as pl


def kernel(x, y, batch, g, mu, sigma, root_w, bias):
    raise NotImplementedError("write your pallas kernel here")


# speedup vs baseline: 2.6170x; 2.6170x over previous
import functools

import jax
import jax.numpy as jnp
import numpy as np
from jax.experimental import pallas as pl

_R = 128
_KNN = 32
_EPS = 1e-15
_BIG = 1e30


def _gmm_knn_kernel(Np, xb_ref, xt_ref, xf_ref, bb_ref, bc_ref, mu_ref,
                    is2_ref, gv_ref, km_ref, rt_ref, bi_ref, out_ref):
    i = pl.program_id(0)
    x_i = xb_ref[...]
    xt = xt_ref[...]
    xf = xf_ref[...]
    sq_i = jnp.sum(x_i * x_i, axis=1, keepdims=True)
    sq_c = jnp.sum(xt * xt, axis=0, keepdims=True)
    prod = jnp.dot(x_i, xt, preferred_element_type=jnp.float32)
    d = sq_i + sq_c - 2.0 * prod
    col = jax.lax.broadcasted_iota(jnp.int32, (_R, Np), 1)
    rowg = i * _R + jax.lax.broadcasted_iota(jnp.int32, (_R, Np), 0)
    valid = (bb_ref[...] == bc_ref[...]) & (col != rowg)
    d = jnp.where(valid, d, _BIG)
    mu3 = mu_ref[...]
    is2 = is2_ref[...]
    gvv = gv_ref[...]
    km = km_ref[...]
    inf = jnp.float32(jnp.inf)

    def body(_, carry):
        d, acc = carry
        m = jnp.min(d, axis=1, keepdims=True)
        sel = d == m
        idx = jnp.min(jnp.where(sel, col, Np), axis=1, keepdims=True)
        oh = col == idx
        x_j = jnp.dot(oh.astype(jnp.float32), xf,
                      preferred_element_type=jnp.float32)
        d = jnp.where(oh, inf, d)
        a = x_j - x_i
        gs = jnp.zeros((_R, 72), jnp.float32)
        for dd in range(3):
            diff = a[:, dd:dd + 1] - mu3[dd:dd + 1, :]
            gs = gs - diff * diff * is2[dd:dd + 1, :]
        w = jnp.exp(gs) * gvv
        w9 = jnp.dot(w, km, preferred_element_type=jnp.float32)
        msg = (x_j[:, 0:1] * w9[:, 0:3] + x_j[:, 1:2] * w9[:, 3:6]
               + x_j[:, 2:3] * w9[:, 6:9])
        return d, acc + msg

    _, acc = jax.lax.fori_loop(0, _KNN, body,
                               (d, jnp.zeros((_R, 3), jnp.float32)))
    out_ref[...] = (acc * (1.0 / _KNN)
                    + jnp.dot(x_i, rt_ref[...],
                              preferred_element_type=jnp.float32)
                    + bi_ref[...])


def _loss_kernel(o_ref, y_ref, out_ref):
    diff = o_ref[...] - y_ref[...]
    out_ref[...] = jnp.sum(diff * diff, keepdims=True).reshape(1, 1)


@jax.jit
def kernel(x, y, batch, g, mu, sigma, root_w, bias):
    n, f = x.shape
    np_pad = ((n + _R - 1) // _R) * _R
    x_pad = jnp.pad(x, ((0, np_pad - n), (0, 0)))
    b32 = batch.astype(jnp.int32)
    b_pad = jnp.pad(b32, (0, np_pad - n), constant_values=-1)
    xt = x_pad.T
    b_rows = b_pad.reshape(np_pad, 1)
    b_cols = b_pad.reshape(1, np_pad)
    f_in, m_dim, k_dim, d_dim = mu.shape
    t = f_in * m_dim * k_dim
    mu3 = mu.reshape(t, d_dim).T
    is2 = (0.5 / (_EPS + sigma * sigma)).reshape(t, d_dim).T
    gv = g.reshape(1, t)
    km = jnp.asarray(np.repeat(np.eye(f_in * m_dim, dtype=np.float32),
                               k_dim, axis=0))
    rt = root_w.T
    bi = bias.reshape(1, f)

    grid = (np_pad // _R,)
    out = pl.pallas_call(
        functools.partial(_gmm_knn_kernel, np_pad),
        grid=grid,
        in_specs=[
            pl.BlockSpec((_R, f), lambda i: (i, 0)),
            pl.BlockSpec((f, np_pad), lambda i: (0, 0)),
            pl.BlockSpec((np_pad, f), lambda i: (0, 0)),
            pl.BlockSpec((_R, 1), lambda i: (i, 0)),
            pl.BlockSpec((1, np_pad), lambda i: (0, 0)),
            pl.BlockSpec((d_dim, t), lambda i: (0, 0)),
            pl.BlockSpec((d_dim, t), lambda i: (0, 0)),
            pl.BlockSpec((1, t), lambda i: (0, 0)),
            pl.BlockSpec((t, f_in * m_dim), lambda i: (0, 0)),
            pl.BlockSpec((f, f), lambda i: (0, 0)),
            pl.BlockSpec((1, f), lambda i: (0, 0)),
        ],
        out_specs=pl.BlockSpec((_R, f), lambda i: (i, 0)),
        out_shape=jax.ShapeDtypeStruct((np_pad, f), jnp.float32),
    )(x_pad, xt, x_pad, b_rows, b_cols, mu3, is2, gv, km, rt, bi)

    out = out[:n]
    sq = pl.pallas_call(
        _loss_kernel,
        out_shape=jax.ShapeDtypeStruct((1, 1), jnp.float32),
    )(out, y)
    loss = sq[0, 0] / (n * f)
    return out, loss


# speedup vs baseline: 6.0325x; 2.3052x over previous
import functools

import jax
import jax.numpy as jnp
import numpy as np
from jax.experimental import pallas as pl
from jax.experimental.pallas import tpu as pltpu

_R = 128
_W = 3328
_KNN = 32
_EPS = 1e-15
_BIG = 1e30


def _select_and_message(x_i, d, col, sentinel, xf, mu3, is2, gvv, km):
    inf = jnp.float32(jnp.inf)

    def body(_, carry):
        d, acc = carry
        m = jnp.min(d, axis=1, keepdims=True)
        sel = d == m
        idx = jnp.min(jnp.where(sel, col, sentinel), axis=1, keepdims=True)
        oh = col == idx
        x_j = jnp.dot(oh.astype(jnp.float32), xf,
                      preferred_element_type=jnp.float32)
        d = jnp.where(oh, inf, d)
        a = x_j - x_i
        gs = jnp.zeros((_R, 72), jnp.float32)
        for dd in range(3):
            diff = a[:, dd:dd + 1] - mu3[dd:dd + 1, :]
            gs = gs - diff * diff * is2[dd:dd + 1, :]
        w = jnp.exp(gs) * gvv
        w9 = jnp.dot(w, km, preferred_element_type=jnp.float32)
        msg = (x_j[:, 0:1] * w9[:, 0:3] + x_j[:, 1:2] * w9[:, 3:6]
               + x_j[:, 2:3] * w9[:, 6:9])
        return d, acc + msg

    _, acc = jax.lax.fori_loop(0, _KNN, body,
                               (d, jnp.zeros((_R, 3), jnp.float32)))
    return acc


def _epilogue(acc, x_i, rt_ref, bi_ref):
    return (acc * (1.0 / _KNN)
            + jnp.dot(x_i, rt_ref[...], preferred_element_type=jnp.float32)
            + bi_ref[...])


def _full_kernel(Np, xb_ref, xt_ref, xf_ref, bb_ref, bc_ref, mu_ref,
                 is2_ref, gv_ref, km_ref, rt_ref, bi_ref, out_ref):
    i = pl.program_id(0)
    x_i = xb_ref[...]
    xt = xt_ref[...]
    sq_i = jnp.sum(x_i * x_i, axis=1, keepdims=True)
    sq_c = jnp.sum(xt * xt, axis=0, keepdims=True)
    prod = jnp.dot(x_i, xt, preferred_element_type=jnp.float32)
    d = sq_i + sq_c - 2.0 * prod
    col = jax.lax.broadcasted_iota(jnp.int32, (_R, Np), 1)
    rowg = i * _R + jax.lax.broadcasted_iota(jnp.int32, (_R, Np), 0)
    valid = (bb_ref[...] == bc_ref[...]) & (col != rowg)
    d = jnp.where(valid, d, _BIG)
    acc = _select_and_message(x_i, d, col, Np, xf_ref[...],
                              mu_ref[...], is2_ref[...], gv_ref[...],
                              km_ref[...])
    out_ref[...] = _epilogue(acc, x_i, rt_ref, bi_ref)


def _win_kernel(Np, W, s_ref, xb_ref, xt_ref, xf_ref, bb_ref, bc_ref, mu_ref,
                is2_ref, gv_ref, km_ref, rt_ref, bi_ref, out_ref):
    i = pl.program_id(0)
    s = pl.multiple_of(s_ref[i], 128)
    x_i = xb_ref[...]
    xt = xt_ref[:, pl.ds(s, W)]
    xf = xf_ref[pl.ds(s, W), :]
    bc = bc_ref[:, pl.ds(s, W)]
    sq_i = jnp.sum(x_i * x_i, axis=1, keepdims=True)
    sq_c = jnp.sum(xt * xt, axis=0, keepdims=True)
    prod = jnp.dot(x_i, xt, preferred_element_type=jnp.float32)
    d = sq_i + sq_c - 2.0 * prod
    col = s + jax.lax.broadcasted_iota(jnp.int32, (_R, W), 1)
    rowg = i * _R + jax.lax.broadcasted_iota(jnp.int32, (_R, W), 0)
    valid = (bb_ref[...] == bc) & (col != rowg)
    d = jnp.where(valid, d, _BIG)
    acc = _select_and_message(x_i, d, col, Np, xf,
                              mu_ref[...], is2_ref[...], gv_ref[...],
                              km_ref[...])
    out_ref[...] = _epilogue(acc, x_i, rt_ref, bi_ref)


def _loss_kernel(o_ref, y_ref, out_ref):
    diff = o_ref[...] - y_ref[...]
    out_ref[...] = jnp.sum(diff * diff, keepdims=True).reshape(1, 1)


@jax.jit
def kernel(x, y, batch, g, mu, sigma, root_w, bias):
    n, f = x.shape
    np_pad = ((n + _R - 1) // _R) * _R
    nblk = np_pad // _R
    x_pad = jnp.pad(x, ((0, np_pad - n), (0, 0)))
    b32 = batch.astype(jnp.int32)
    b_pad = jnp.pad(b32, (0, np_pad - n), constant_values=-1)
    xt = x_pad.T
    b_rows = b_pad.reshape(np_pad, 1)
    b_cols = b_pad.reshape(1, np_pad)
    f_in, m_dim, k_dim, d_dim = mu.shape
    t = f_in * m_dim * k_dim
    mu3 = mu.reshape(t, d_dim).T
    is2 = (0.5 / (_EPS + sigma * sigma)).reshape(t, d_dim).T
    gv = g.reshape(1, t)
    km = jnp.asarray(np.repeat(np.eye(f_in * m_dim, dtype=np.float32),
                               k_dim, axis=0))
    rt = root_w.T
    bi = bias.reshape(1, f)

    first_b = b_pad[::_R]
    last_b = b32[jnp.minimum(jnp.arange(nblk) * _R + _R - 1, n - 1)]
    weff = min(_W, np_pad)
    starts = jnp.searchsorted(b32, first_b, side="left").astype(jnp.int32)
    ends = jnp.searchsorted(b32, last_b, side="right").astype(jnp.int32)
    starts = jnp.minimum((starts // 128) * 128, np_pad - weff)
    fits = jnp.max(ends - starts) <= weff
    vals = jnp.arange(8, dtype=jnp.int32)
    cnt = (jnp.searchsorted(b32, vals, side="right")
           - jnp.searchsorted(b32, vals, side="left"))
    seg_ok = jnp.min(jnp.where(cnt > 0, cnt, 33)) >= 33
    use_win = fits & seg_ok

    def make_ins(blocked):
        if blocked:
            blk = lambda: (lambda i: (i, 0))
            whole = lambda: (lambda i: (0, 0))
        else:
            blk = lambda: (lambda i, s: (i, 0))
            whole = lambda: (lambda i, s: (0, 0))
        return [
            pl.BlockSpec((_R, f), blk()),
            pl.BlockSpec((f, np_pad), whole()),
            pl.BlockSpec((np_pad, f), whole()),
            pl.BlockSpec((_R, 1), blk()),
            pl.BlockSpec((1, np_pad), whole()),
            pl.BlockSpec((d_dim, t), whole()),
            pl.BlockSpec((d_dim, t), whole()),
            pl.BlockSpec((1, t), whole()),
            pl.BlockSpec((t, f_in * m_dim), whole()),
            pl.BlockSpec((f, f), whole()),
            pl.BlockSpec((1, f), whole()),
        ]

    dense_args = (x_pad, xt, x_pad, b_rows, b_cols, mu3, is2, gv, km, rt, bi)
    out_sds = jax.ShapeDtypeStruct((np_pad, f), jnp.float32)

    def run_win(_):
        return pl.pallas_call(
            functools.partial(_win_kernel, np_pad, weff),
            grid_spec=pltpu.PrefetchScalarGridSpec(
                num_scalar_prefetch=1,
                grid=(nblk,),
                in_specs=make_ins(False),
                out_specs=pl.BlockSpec((_R, f), lambda i, s: (i, 0)),
            ),
            out_shape=out_sds,
        )(starts, *dense_args)

    def run_full(_):
        return pl.pallas_call(
            functools.partial(_full_kernel, np_pad),
            grid=(nblk,),
            in_specs=make_ins(True),
            out_specs=pl.BlockSpec((_R, f), lambda i: (i, 0)),
            out_shape=out_sds,
        )(*dense_args)

    out = jax.lax.cond(use_win, run_win, run_full, operand=None)

    out = out[:n]
    sq = pl.pallas_call(
        _loss_kernel,
        out_shape=jax.ShapeDtypeStruct((1, 1), jnp.float32),
    )(out, y)
    loss = sq[0, 0] / (n * f)
    return out, loss


# speedup vs baseline: 6.2389x; 1.0342x over previous
import functools

import jax
import jax.numpy as jnp
import numpy as np
from jax.experimental import pallas as pl
from jax.experimental.pallas import tpu as pltpu

_R = 128
_W = 3328
_KNN = 32
_EPS = 1e-15
_BIG = 1e30


def _select_and_message(x_i, d, col, xf, mu3, is2, gvv, km):
    inf = jnp.float32(jnp.inf)

    def body(_, carry):
        d, acc = carry
        idx = jnp.argmin(d, axis=1).astype(jnp.int32).reshape(_R, 1)
        oh = col == idx
        x_j = jnp.dot(oh.astype(jnp.float32), xf,
                      preferred_element_type=jnp.float32)
        d = jnp.where(oh, inf, d)
        a = x_j - x_i
        gs = jnp.zeros((_R, 72), jnp.float32)
        for dd in range(3):
            diff = a[:, dd:dd + 1] - mu3[dd:dd + 1, :]
            gs = gs - diff * diff * is2[dd:dd + 1, :]
        w = jnp.exp(gs) * gvv
        w9 = jnp.dot(w, km, preferred_element_type=jnp.float32)
        msg = (x_j[:, 0:1] * w9[:, 0:3] + x_j[:, 1:2] * w9[:, 3:6]
               + x_j[:, 2:3] * w9[:, 6:9])
        return d, acc + msg

    _, acc = jax.lax.fori_loop(0, _KNN, body,
                               (d, jnp.zeros((_R, 3), jnp.float32)))
    return acc


def _epilogue(acc, x_i, rt_ref, bi_ref):
    return (acc * (1.0 / _KNN)
            + jnp.dot(x_i, rt_ref[...], preferred_element_type=jnp.float32)
            + bi_ref[...])


def _full_kernel(Np, xb_ref, xt_ref, xf_ref, bb_ref, bc_ref, mu_ref,
                 is2_ref, gv_ref, km_ref, rt_ref, bi_ref, out_ref):
    i = pl.program_id(0)
    x_i = xb_ref[...]
    xt = xt_ref[...]
    sq_i = jnp.sum(x_i * x_i, axis=1, keepdims=True)
    sq_c = jnp.sum(xt * xt, axis=0, keepdims=True)
    prod = jnp.dot(x_i, xt, preferred_element_type=jnp.float32)
    d = sq_i + sq_c - 2.0 * prod
    col = jax.lax.broadcasted_iota(jnp.int32, (_R, Np), 1)
    rowg = i * _R + jax.lax.broadcasted_iota(jnp.int32, (_R, Np), 0)
    valid = (bb_ref[...] == bc_ref[...]) & (col != rowg)
    d = jnp.where(valid, d, _BIG)
    acc = _select_and_message(x_i, d, col, xf_ref[...],
                              mu_ref[...], is2_ref[...], gv_ref[...],
                              km_ref[...])
    out_ref[...] = _epilogue(acc, x_i, rt_ref, bi_ref)


def _win_kernel(Np, W, s_ref, xb_ref, xt_ref, xf_ref, bb_ref, bc_ref, mu_ref,
                is2_ref, gv_ref, km_ref, rt_ref, bi_ref, out_ref):
    i = pl.program_id(0)
    s = pl.multiple_of(s_ref[i], 128)
    x_i = xb_ref[...]
    xt = xt_ref[:, pl.ds(s, W)]
    xf = xf_ref[pl.ds(s, W), :]
    bc = bc_ref[:, pl.ds(s, W)]
    sq_i = jnp.sum(x_i * x_i, axis=1, keepdims=True)
    sq_c = jnp.sum(xt * xt, axis=0, keepdims=True)
    prod = jnp.dot(x_i, xt, preferred_element_type=jnp.float32)
    d = sq_i + sq_c - 2.0 * prod
    lcol = jax.lax.broadcasted_iota(jnp.int32, (_R, W), 1)
    rowg = i * _R + jax.lax.broadcasted_iota(jnp.int32, (_R, W), 0)
    valid = (bb_ref[...] == bc) & (s + lcol != rowg)
    d = jnp.where(valid, d, _BIG)
    acc = _select_and_message(x_i, d, lcol, xf,
                              mu_ref[...], is2_ref[...], gv_ref[...],
                              km_ref[...])
    out_ref[...] = _epilogue(acc, x_i, rt_ref, bi_ref)


def _loss_kernel(o_ref, y_ref, out_ref):
    diff = o_ref[...] - y_ref[...]
    out_ref[...] = jnp.sum(diff * diff, keepdims=True).reshape(1, 1)


@jax.jit
def kernel(x, y, batch, g, mu, sigma, root_w, bias):
    n, f = x.shape
    np_pad = ((n + _R - 1) // _R) * _R
    nblk = np_pad // _R
    x_pad = jnp.pad(x, ((0, np_pad - n), (0, 0)))
    b32 = batch.astype(jnp.int32)
    b_pad = jnp.pad(b32, (0, np_pad - n), constant_values=-1)
    xt = x_pad.T
    b_rows = b_pad.reshape(np_pad, 1)
    b_cols = b_pad.reshape(1, np_pad)
    f_in, m_dim, k_dim, d_dim = mu.shape
    t = f_in * m_dim * k_dim
    mu3 = mu.reshape(t, d_dim).T
    is2 = (0.5 / (_EPS + sigma * sigma)).reshape(t, d_dim).T
    gv = g.reshape(1, t)
    km = jnp.asarray(np.repeat(np.eye(f_in * m_dim, dtype=np.float32),
                               k_dim, axis=0))
    rt = root_w.T
    bi = bias.reshape(1, f)

    first_b = b_pad[::_R]
    last_b = b32[jnp.minimum(jnp.arange(nblk) * _R + _R - 1, n - 1)]
    weff = min(_W, np_pad)
    starts = jnp.searchsorted(b32, first_b, side="left").astype(jnp.int32)
    ends = jnp.searchsorted(b32, last_b, side="right").astype(jnp.int32)
    starts = jnp.minimum((starts // 128) * 128, np_pad - weff)
    fits = jnp.max(ends - starts) <= weff
    vals = jnp.arange(8, dtype=jnp.int32)
    cnt = (jnp.searchsorted(b32, vals, side="right")
           - jnp.searchsorted(b32, vals, side="left"))
    seg_ok = jnp.min(jnp.where(cnt > 0, cnt, 33)) >= 33
    use_win = fits & seg_ok

    def make_ins(blocked):
        if blocked:
            blk = lambda: (lambda i: (i, 0))
            whole = lambda: (lambda i: (0, 0))
        else:
            blk = lambda: (lambda i, s: (i, 0))
            whole = lambda: (lambda i, s: (0, 0))
        return [
            pl.BlockSpec((_R, f), blk()),
            pl.BlockSpec((f, np_pad), whole()),
            pl.BlockSpec((np_pad, f), whole()),
            pl.BlockSpec((_R, 1), blk()),
            pl.BlockSpec((1, np_pad), whole()),
            pl.BlockSpec((d_dim, t), whole()),
            pl.BlockSpec((d_dim, t), whole()),
            pl.BlockSpec((1, t), whole()),
            pl.BlockSpec((t, f_in * m_dim), whole()),
            pl.BlockSpec((f, f), whole()),
            pl.BlockSpec((1, f), whole()),
        ]

    dense_args = (x_pad, xt, x_pad, b_rows, b_cols, mu3, is2, gv, km, rt, bi)
    out_sds = jax.ShapeDtypeStruct((np_pad, f), jnp.float32)

    def run_win(_):
        return pl.pallas_call(
            functools.partial(_win_kernel, np_pad, weff),
            grid_spec=pltpu.PrefetchScalarGridSpec(
                num_scalar_prefetch=1,
                grid=(nblk,),
                in_specs=make_ins(False),
                out_specs=pl.BlockSpec((_R, f), lambda i, s: (i, 0)),
            ),
            out_shape=out_sds,
        )(starts, *dense_args)

    def run_full(_):
        return pl.pallas_call(
            functools.partial(_full_kernel, np_pad),
            grid=(nblk,),
            in_specs=make_ins(True),
            out_specs=pl.BlockSpec((_R, f), lambda i: (i, 0)),
            out_shape=out_sds,
        )(*dense_args)

    out = jax.lax.cond(use_win, run_win, run_full, operand=None)

    out = out[:n]
    sq = pl.pallas_call(
        _loss_kernel,
        out_shape=jax.ShapeDtypeStruct((1, 1), jnp.float32),
    )(out, y)
    loss = sq[0, 0] / (n * f)
    return out, loss


# speedup vs baseline: 8.6049x; 1.3792x over previous
import functools

import jax
import jax.numpy as jnp
import numpy as np
from jax.experimental import pallas as pl
from jax.experimental.pallas import tpu as pltpu

_R = 128
_W = 3328
_W2 = 1664
_KNN = 32
_EPS = 1e-15
_BIG = 1e30


def _select_and_message(x_i, d, col, xf, mu3, is2, gvv, km):
    inf = jnp.float32(jnp.inf)

    def body(_, carry):
        d, acc = carry
        idx = jnp.argmin(d, axis=1).astype(jnp.int32).reshape(_R, 1)
        oh = col == idx
        x_j = jnp.dot(oh.astype(jnp.float32), xf,
                      preferred_element_type=jnp.float32)
        d = jnp.where(oh, inf, d)
        a = x_j - x_i
        gs = jnp.zeros((_R, 72), jnp.float32)
        for dd in range(3):
            diff = a[:, dd:dd + 1] - mu3[dd:dd + 1, :]
            gs = gs - diff * diff * is2[dd:dd + 1, :]
        w = jnp.exp(gs) * gvv
        w9 = jnp.dot(w, km, preferred_element_type=jnp.float32)
        msg = (x_j[:, 0:1] * w9[:, 0:3] + x_j[:, 1:2] * w9[:, 3:6]
               + x_j[:, 2:3] * w9[:, 6:9])
        return d, acc + msg

    _, acc = jax.lax.fori_loop(0, _KNN, body,
                               (d, jnp.zeros((_R, 3), jnp.float32)))
    return acc


def _epilogue(acc, x_i, rt_ref, bi_ref):
    return (acc * (1.0 / _KNN)
            + jnp.dot(x_i, rt_ref[...], preferred_element_type=jnp.float32)
            + bi_ref[...])


def _full_kernel(Np, xb_ref, xt_ref, xf_ref, bb_ref, bc_ref, mu_ref,
                 is2_ref, gv_ref, km_ref, rt_ref, bi_ref, out_ref):
    i = pl.program_id(0)
    x_i = xb_ref[...]
    xt = xt_ref[...]
    sq_i = jnp.sum(x_i * x_i, axis=1, keepdims=True)
    sq_c = jnp.sum(xt * xt, axis=0, keepdims=True)
    prod = jnp.dot(x_i, xt, preferred_element_type=jnp.float32)
    d = sq_i + sq_c - 2.0 * prod
    col = jax.lax.broadcasted_iota(jnp.int32, (_R, Np), 1)
    rowg = i * _R + jax.lax.broadcasted_iota(jnp.int32, (_R, Np), 0)
    valid = (bb_ref[...] == bc_ref[...]) & (col != rowg)
    d = jnp.where(valid, d, _BIG)
    acc = _select_and_message(x_i, d, col, xf_ref[...],
                              mu_ref[...], is2_ref[...], gv_ref[...],
                              km_ref[...])
    out_ref[...] = _epilogue(acc, x_i, rt_ref, bi_ref)


def _win_kernel(Np, W, s_ref, xb_ref, xt_ref, xf_ref, bb_ref, bc_ref, mu_ref,
                is2_ref, gv_ref, km_ref, rt_ref, bi_ref, out_ref):
    i = pl.program_id(0)
    s = pl.multiple_of(s_ref[i], 128)
    x_i = xb_ref[...]
    xt = xt_ref[:, pl.ds(s, W)]
    xf = xf_ref[pl.ds(s, W), :]
    bc = bc_ref[:, pl.ds(s, W)]
    sq_i = jnp.sum(x_i * x_i, axis=1, keepdims=True)
    sq_c = jnp.sum(xt * xt, axis=0, keepdims=True)
    prod = jnp.dot(x_i, xt, preferred_element_type=jnp.float32)
    d = sq_i + sq_c - 2.0 * prod
    lcol = jax.lax.broadcasted_iota(jnp.int32, (_R, W), 1)
    rowg = i * _R + jax.lax.broadcasted_iota(jnp.int32, (_R, W), 0)
    valid = (bb_ref[...] == bc) & (s + lcol != rowg)
    d = jnp.where(valid, d, _BIG)
    acc = _select_and_message(x_i, d, lcol, xf,
                              mu_ref[...], is2_ref[...], gv_ref[...],
                              km_ref[...])
    out_ref[...] = _epilogue(acc, x_i, rt_ref, bi_ref)


def _seg_kernel(Np, W, rs_ref, cs_ref, a_ref, b_ref, xt_ref, xf_ref, bp_ref,
                bc_ref, mu_ref, is2_ref, gv_ref, km_ref, rt_ref, bi_ref,
                out_ref):
    t = pl.program_id(0)
    a = a_ref[t]
    bnd = b_ref[t]

    @pl.when(bnd > a)
    def _():
        rs = pl.multiple_of(rs_ref[t], 8)
        s = pl.multiple_of(cs_ref[t], 128)
        x_i = xf_ref[pl.ds(rs, _R), :]
        bb = bp_ref[pl.ds(rs, _R), :]
        xt = xt_ref[:, pl.ds(s, W)]
        xf = xf_ref[pl.ds(s, W), :]
        bc = bc_ref[:, pl.ds(s, W)]
        sq_i = jnp.sum(x_i * x_i, axis=1, keepdims=True)
        sq_c = jnp.sum(xt * xt, axis=0, keepdims=True)
        prod = jnp.dot(x_i, xt, preferred_element_type=jnp.float32)
        d = sq_i + sq_c - 2.0 * prod
        lcol = jax.lax.broadcasted_iota(jnp.int32, (_R, W), 1)
        rowg = rs + jax.lax.broadcasted_iota(jnp.int32, (_R, W), 0)
        valid = (bb == bc) & (s + lcol != rowg)
        d = jnp.where(valid, d, _BIG)
        acc = _select_and_message(x_i, d, lcol, xf,
                                  mu_ref[...], is2_ref[...], gv_ref[...],
                                  km_ref[...])
        val = _epilogue(acc, x_i, rt_ref, bi_ref)
        ridx = rs + jax.lax.broadcasted_iota(jnp.int32, (_R, 1), 0)
        rmask = (ridx >= a) & (ridx < bnd)
        cur = out_ref[pl.ds(rs, _R), :]
        out_ref[pl.ds(rs, _R), :] = jnp.where(rmask, val, cur)


def _loss_kernel(o_ref, y_ref, out_ref):
    diff = o_ref[...] - y_ref[...]
    out_ref[...] = jnp.sum(diff * diff, keepdims=True).reshape(1, 1)


@jax.jit
def kernel(x, y, batch, g, mu, sigma, root_w, bias):
    n, f = x.shape
    np_pad = ((n + _R - 1) // _R) * _R
    nblk = np_pad // _R
    x_pad = jnp.pad(x, ((0, np_pad - n), (0, 0)))
    b32 = batch.astype(jnp.int32)
    b_pad = jnp.pad(b32, (0, np_pad - n), constant_values=-1)
    xt = x_pad.T
    b_rows = b_pad.reshape(np_pad, 1)
    b_cols = b_pad.reshape(1, np_pad)
    f_in, m_dim, k_dim, d_dim = mu.shape
    t = f_in * m_dim * k_dim
    mu3 = mu.reshape(t, d_dim).T
    is2 = (0.5 / (_EPS + sigma * sigma)).reshape(t, d_dim).T
    gv = g.reshape(1, t)
    km = jnp.asarray(np.repeat(np.eye(f_in * m_dim, dtype=np.float32),
                               k_dim, axis=0))
    rt = root_w.T
    bi = bias.reshape(1, f)

    first_b = b_pad[::_R]
    last_b = b32[jnp.minimum(jnp.arange(nblk) * _R + _R - 1, n - 1)]
    weff = min(_W, np_pad)
    starts = jnp.searchsorted(b32, first_b, side="left").astype(jnp.int32)
    ends = jnp.searchsorted(b32, last_b, side="right").astype(jnp.int32)
    starts = jnp.minimum((starts // 128) * 128, np_pad - weff)
    fits = jnp.max(ends - starts) <= weff
    vals = jnp.arange(8, dtype=jnp.int32)
    cnt = (jnp.searchsorted(b32, vals, side="right")
           - jnp.searchsorted(b32, vals, side="left"))
    seg_ok = jnp.min(jnp.where(cnt > 0, cnt, 33)) >= 33
    use_win = fits & seg_ok

    w2 = min(_W2, np_pad)
    seg_a = jnp.searchsorted(b32, vals, side="left").astype(jnp.int32)
    seg_b = jnp.searchsorted(b32, vals, side="right").astype(jnp.int32)
    nb8 = (seg_b - seg_a + 7 + _R - 1) // _R
    cum = jnp.cumsum(nb8)
    nbt = nblk + 9
    ts = jnp.arange(nbt, dtype=jnp.int32)
    segix = jnp.clip(jnp.searchsorted(cum, ts, side="right"), 0, 7)
    jj = ts - (cum[segix] - nb8[segix])
    a_t = seg_a[segix]
    b_t = seg_b[segix]
    rs_t = jnp.clip((a_t // 8) * 8 + _R * jj, 0, np_pad - _R).astype(jnp.int32)
    cs8 = jnp.clip((seg_a // 128) * 128, 0, np_pad - w2)
    cs_t = cs8[segix].astype(jnp.int32)
    valid_t = ts < cum[7]
    a_t = jnp.where(valid_t, a_t, 0).astype(jnp.int32)
    b_t = jnp.where(valid_t, b_t, 0).astype(jnp.int32)
    fits2 = jnp.max(jnp.where(cnt > 0, seg_b - cs8, 0)) <= w2
    use_seg = fits2 & seg_ok

    def make_ins(blocked):
        if blocked:
            blk = lambda: (lambda i: (i, 0))
            whole = lambda: (lambda i: (0, 0))
        else:
            blk = lambda: (lambda i, s: (i, 0))
            whole = lambda: (lambda i, s: (0, 0))
        return [
            pl.BlockSpec((_R, f), blk()),
            pl.BlockSpec((f, np_pad), whole()),
            pl.BlockSpec((np_pad, f), whole()),
            pl.BlockSpec((_R, 1), blk()),
            pl.BlockSpec((1, np_pad), whole()),
            pl.BlockSpec((d_dim, t), whole()),
            pl.BlockSpec((d_dim, t), whole()),
            pl.BlockSpec((1, t), whole()),
            pl.BlockSpec((t, f_in * m_dim), whole()),
            pl.BlockSpec((f, f), whole()),
            pl.BlockSpec((1, f), whole()),
        ]

    dense_args = (x_pad, xt, x_pad, b_rows, b_cols, mu3, is2, gv, km, rt, bi)
    out_sds = jax.ShapeDtypeStruct((np_pad, f), jnp.float32)

    def run_win(_):
        return pl.pallas_call(
            functools.partial(_win_kernel, np_pad, weff),
            grid_spec=pltpu.PrefetchScalarGridSpec(
                num_scalar_prefetch=1,
                grid=(nblk,),
                in_specs=make_ins(False),
                out_specs=pl.BlockSpec((_R, f), lambda i, s: (i, 0)),
            ),
            out_shape=out_sds,
        )(starts, *dense_args)

    def run_full(_):
        return pl.pallas_call(
            functools.partial(_full_kernel, np_pad),
            grid=(nblk,),
            in_specs=make_ins(True),
            out_specs=pl.BlockSpec((_R, f), lambda i: (i, 0)),
            out_shape=out_sds,
        )(*dense_args)

    def run_seg(_):
        whole = lambda: (lambda i, *_: (0, 0))
        seg_ins = [
            pl.BlockSpec((f, np_pad), whole()),
            pl.BlockSpec((np_pad, f), whole()),
            pl.BlockSpec((np_pad, 1), whole()),
            pl.BlockSpec((1, np_pad), whole()),
            pl.BlockSpec((d_dim, t), whole()),
            pl.BlockSpec((d_dim, t), whole()),
            pl.BlockSpec((1, t), whole()),
            pl.BlockSpec((t, f_in * m_dim), whole()),
            pl.BlockSpec((f, f), whole()),
            pl.BlockSpec((1, f), whole()),
        ]
        return pl.pallas_call(
            functools.partial(_seg_kernel, np_pad, w2),
            grid_spec=pltpu.PrefetchScalarGridSpec(
                num_scalar_prefetch=4,
                grid=(nbt,),
                in_specs=seg_ins,
                out_specs=pl.BlockSpec((np_pad, f), lambda i, *_: (0, 0)),
            ),
            out_shape=out_sds,
        )(rs_t, cs_t, a_t, b_t, xt, x_pad, b_rows, b_cols, mu3, is2, gv,
          km, rt, bi)

    out = jax.lax.cond(
        use_seg, run_seg,
        lambda _: jax.lax.cond(use_win, run_win, run_full, operand=None),
        operand=None)

    out = out[:n]
    sq = pl.pallas_call(
        _loss_kernel,
        out_shape=jax.ShapeDtypeStruct((1, 1), jnp.float32),
    )(out, y)
    loss = sq[0, 0] / (n * f)
    return out, loss


# speedup vs baseline: 9.7462x; 1.1326x over previous
import functools

import jax
import jax.numpy as jnp
import numpy as np
from jax.experimental import pallas as pl
from jax.experimental.pallas import tpu as pltpu

_R = 256
_W = 3328
_W2 = 1664
_KNN = 32
_EPS = 1e-15
_BIG = 1e30


def _select_and_message(x_i, d, col, xf, mu3, is2, gvv, km):
    inf = jnp.float32(jnp.inf)

    def body(_, carry):
        d, acc = carry
        idx = jnp.argmin(d, axis=1).astype(jnp.int32).reshape(_R, 1)
        oh = col == idx
        x_j = jnp.dot(oh.astype(jnp.float32), xf,
                      preferred_element_type=jnp.float32)
        d = jnp.where(oh, inf, d)
        a = x_j - x_i
        gs = jnp.zeros((_R, 72), jnp.float32)
        for dd in range(3):
            diff = a[:, dd:dd + 1] - mu3[dd:dd + 1, :]
            gs = gs - diff * diff * is2[dd:dd + 1, :]
        w = jnp.exp(gs) * gvv
        w9 = jnp.dot(w, km, preferred_element_type=jnp.float32)
        msg = (x_j[:, 0:1] * w9[:, 0:3] + x_j[:, 1:2] * w9[:, 3:6]
               + x_j[:, 2:3] * w9[:, 6:9])
        return d, acc + msg

    _, acc = jax.lax.fori_loop(0, _KNN, body,
                               (d, jnp.zeros((_R, 3), jnp.float32)))
    return acc


def _epilogue(acc, x_i, rt_ref, bi_ref):
    return (acc * (1.0 / _KNN)
            + jnp.dot(x_i, rt_ref[...], preferred_element_type=jnp.float32)
            + bi_ref[...])


def _full_kernel(Np, xb_ref, xt_ref, xf_ref, bb_ref, bc_ref, mu_ref,
                 is2_ref, gv_ref, km_ref, rt_ref, bi_ref, out_ref):
    i = pl.program_id(0)
    x_i = xb_ref[...]
    xt = xt_ref[...]
    sq_i = jnp.sum(x_i * x_i, axis=1, keepdims=True)
    sq_c = jnp.sum(xt * xt, axis=0, keepdims=True)
    prod = jnp.dot(x_i, xt, preferred_element_type=jnp.float32)
    d = sq_i + sq_c - 2.0 * prod
    col = jax.lax.broadcasted_iota(jnp.int32, (_R, Np), 1)
    rowg = i * _R + jax.lax.broadcasted_iota(jnp.int32, (_R, Np), 0)
    valid = (bb_ref[...] == bc_ref[...]) & (col != rowg)
    d = jnp.where(valid, d, _BIG)
    acc = _select_and_message(x_i, d, col, xf_ref[...],
                              mu_ref[...], is2_ref[...], gv_ref[...],
                              km_ref[...])
    out_ref[...] = _epilogue(acc, x_i, rt_ref, bi_ref)


def _win_kernel(Np, W, s_ref, xb_ref, xt_ref, xf_ref, bb_ref, bc_ref, mu_ref,
                is2_ref, gv_ref, km_ref, rt_ref, bi_ref, out_ref):
    i = pl.program_id(0)
    s = pl.multiple_of(s_ref[i], 128)
    x_i = xb_ref[...]
    xt = xt_ref[:, pl.ds(s, W)]
    xf = xf_ref[pl.ds(s, W), :]
    bc = bc_ref[:, pl.ds(s, W)]
    sq_i = jnp.sum(x_i * x_i, axis=1, keepdims=True)
    sq_c = jnp.sum(xt * xt, axis=0, keepdims=True)
    prod = jnp.dot(x_i, xt, preferred_element_type=jnp.float32)
    d = sq_i + sq_c - 2.0 * prod
    lcol = jax.lax.broadcasted_iota(jnp.int32, (_R, W), 1)
    rowg = i * _R + jax.lax.broadcasted_iota(jnp.int32, (_R, W), 0)
    valid = (bb_ref[...] == bc) & (s + lcol != rowg)
    d = jnp.where(valid, d, _BIG)
    acc = _select_and_message(x_i, d, lcol, xf,
                              mu_ref[...], is2_ref[...], gv_ref[...],
                              km_ref[...])
    out_ref[...] = _epilogue(acc, x_i, rt_ref, bi_ref)


def _seg_kernel(Np, W, rs_ref, cs_ref, a_ref, b_ref, xt_ref, xf_ref, bp_ref,
                bc_ref, mu_ref, is2_ref, gv_ref, km_ref, rt_ref, bi_ref,
                out_ref):
    t = pl.program_id(0)
    a = a_ref[t]
    bnd = b_ref[t]

    @pl.when(bnd > a)
    def _():
        rs = pl.multiple_of(rs_ref[t], 8)
        s = pl.multiple_of(cs_ref[t], 128)
        x_i = xf_ref[pl.ds(rs, _R), :]
        bb = bp_ref[pl.ds(rs, _R), :]
        xt = xt_ref[:, pl.ds(s, W)]
        xf = xf_ref[pl.ds(s, W), :]
        bc = bc_ref[:, pl.ds(s, W)]
        sq_i = jnp.sum(x_i * x_i, axis=1, keepdims=True)
        sq_c = jnp.sum(xt * xt, axis=0, keepdims=True)
        prod = jnp.dot(x_i, xt, preferred_element_type=jnp.float32)
        d = sq_i + sq_c - 2.0 * prod
        lcol = jax.lax.broadcasted_iota(jnp.int32, (_R, W), 1)
        rowg = rs + jax.lax.broadcasted_iota(jnp.int32, (_R, W), 0)
        valid = (bb == bc) & (s + lcol != rowg)
        d = jnp.where(valid, d, _BIG)
        acc = _select_and_message(x_i, d, lcol, xf,
                                  mu_ref[...], is2_ref[...], gv_ref[...],
                                  km_ref[...])
        val = _epilogue(acc, x_i, rt_ref, bi_ref)
        ridx = rs + jax.lax.broadcasted_iota(jnp.int32, (_R, 1), 0)
        rmask = (ridx >= a) & (ridx < bnd)
        cur = out_ref[pl.ds(rs, _R), :]
        out_ref[pl.ds(rs, _R), :] = jnp.where(rmask, val, cur)


def _loss_kernel(o_ref, y_ref, out_ref):
    diff = o_ref[...] - y_ref[...]
    out_ref[...] = jnp.sum(diff * diff, keepdims=True).reshape(1, 1)


@jax.jit
def kernel(x, y, batch, g, mu, sigma, root_w, bias):
    n, f = x.shape
    np_pad = ((n + _R - 1) // _R) * _R
    nblk = np_pad // _R
    x_pad = jnp.pad(x, ((0, np_pad - n), (0, 0)))
    b32 = batch.astype(jnp.int32)
    b_pad = jnp.pad(b32, (0, np_pad - n), constant_values=-1)
    xt = x_pad.T
    b_rows = b_pad.reshape(np_pad, 1)
    b_cols = b_pad.reshape(1, np_pad)
    f_in, m_dim, k_dim, d_dim = mu.shape
    t = f_in * m_dim * k_dim
    mu3 = mu.reshape(t, d_dim).T
    is2 = (0.5 / (_EPS + sigma * sigma)).reshape(t, d_dim).T
    gv = g.reshape(1, t)
    km = jnp.asarray(np.repeat(np.eye(f_in * m_dim, dtype=np.float32),
                               k_dim, axis=0))
    rt = root_w.T
    bi = bias.reshape(1, f)

    first_b = b_pad[::_R]
    last_b = b32[jnp.minimum(jnp.arange(nblk) * _R + _R - 1, n - 1)]
    weff = min(_W, np_pad)
    starts = jnp.searchsorted(b32, first_b, side="left").astype(jnp.int32)
    ends = jnp.searchsorted(b32, last_b, side="right").astype(jnp.int32)
    starts = jnp.minimum((starts // 128) * 128, np_pad - weff)
    fits = jnp.max(ends - starts) <= weff
    vals = jnp.arange(8, dtype=jnp.int32)
    cnt = (jnp.searchsorted(b32, vals, side="right")
           - jnp.searchsorted(b32, vals, side="left"))
    seg_ok = jnp.min(jnp.where(cnt > 0, cnt, 33)) >= 33
    use_win = fits & seg_ok

    w2 = min(_W2, np_pad)
    seg_a = jnp.searchsorted(b32, vals, side="left").astype(jnp.int32)
    seg_b = jnp.searchsorted(b32, vals, side="right").astype(jnp.int32)
    nb8 = (seg_b - seg_a + 7 + _R - 1) // _R
    cum = jnp.cumsum(nb8)
    nbt = nblk + 9
    ts = jnp.arange(nbt, dtype=jnp.int32)
    segix = jnp.clip(jnp.searchsorted(cum, ts, side="right"), 0, 7)
    jj = ts - (cum[segix] - nb8[segix])
    a_t = seg_a[segix]
    b_t = seg_b[segix]
    rs_t = jnp.clip((a_t // 8) * 8 + _R * jj, 0, np_pad - _R).astype(jnp.int32)
    cs8 = jnp.clip((seg_a // 128) * 128, 0, np_pad - w2)
    cs_t = cs8[segix].astype(jnp.int32)
    valid_t = ts < cum[7]
    a_t = jnp.where(valid_t, a_t, 0).astype(jnp.int32)
    b_t = jnp.where(valid_t, b_t, 0).astype(jnp.int32)
    fits2 = jnp.max(jnp.where(cnt > 0, seg_b - cs8, 0)) <= w2
    use_seg = fits2 & seg_ok

    def make_ins(blocked):
        if blocked:
            blk = lambda: (lambda i: (i, 0))
            whole = lambda: (lambda i: (0, 0))
        else:
            blk = lambda: (lambda i, s: (i, 0))
            whole = lambda: (lambda i, s: (0, 0))
        return [
            pl.BlockSpec((_R, f), blk()),
            pl.BlockSpec((f, np_pad), whole()),
            pl.BlockSpec((np_pad, f), whole()),
            pl.BlockSpec((_R, 1), blk()),
            pl.BlockSpec((1, np_pad), whole()),
            pl.BlockSpec((d_dim, t), whole()),
            pl.BlockSpec((d_dim, t), whole()),
            pl.BlockSpec((1, t), whole()),
            pl.BlockSpec((t, f_in * m_dim), whole()),
            pl.BlockSpec((f, f), whole()),
            pl.BlockSpec((1, f), whole()),
        ]

    dense_args = (x_pad, xt, x_pad, b_rows, b_cols, mu3, is2, gv, km, rt, bi)
    out_sds = jax.ShapeDtypeStruct((np_pad, f), jnp.float32)

    def run_win(_):
        return pl.pallas_call(
            functools.partial(_win_kernel, np_pad, weff),
            grid_spec=pltpu.PrefetchScalarGridSpec(
                num_scalar_prefetch=1,
                grid=(nblk,),
                in_specs=make_ins(False),
                out_specs=pl.BlockSpec((_R, f), lambda i, s: (i, 0)),
            ),
            out_shape=out_sds,
        )(starts, *dense_args)

    def run_full(_):
        return pl.pallas_call(
            functools.partial(_full_kernel, np_pad),
            grid=(nblk,),
            in_specs=make_ins(True),
            out_specs=pl.BlockSpec((_R, f), lambda i: (i, 0)),
            out_shape=out_sds,
        )(*dense_args)

    def run_seg(_):
        whole = lambda: (lambda i, *_: (0, 0))
        seg_ins = [
            pl.BlockSpec((f, np_pad), whole()),
            pl.BlockSpec((np_pad, f), whole()),
            pl.BlockSpec((np_pad, 1), whole()),
            pl.BlockSpec((1, np_pad), whole()),
            pl.BlockSpec((d_dim, t), whole()),
            pl.BlockSpec((d_dim, t), whole()),
            pl.BlockSpec((1, t), whole()),
            pl.BlockSpec((t, f_in * m_dim), whole()),
            pl.BlockSpec((f, f), whole()),
            pl.BlockSpec((1, f), whole()),
        ]
        return pl.pallas_call(
            functools.partial(_seg_kernel, np_pad, w2),
            grid_spec=pltpu.PrefetchScalarGridSpec(
                num_scalar_prefetch=4,
                grid=(nbt,),
                in_specs=seg_ins,
                out_specs=pl.BlockSpec((np_pad, f), lambda i, *_: (0, 0)),
            ),
            out_shape=out_sds,
        )(rs_t, cs_t, a_t, b_t, xt, x_pad, b_rows, b_cols, mu3, is2, gv,
          km, rt, bi)

    out = jax.lax.cond(
        use_seg, run_seg,
        lambda _: jax.lax.cond(use_win, run_win, run_full, operand=None),
        operand=None)

    out = out[:n]
    sq = pl.pallas_call(
        _loss_kernel,
        out_shape=jax.ShapeDtypeStruct((1, 1), jnp.float32),
    )(out, y)
    loss = sq[0, 0] / (n * f)
    return out, loss


# speedup vs baseline: 10.9271x; 1.1212x over previous
import functools

import jax
import jax.numpy as jnp
import numpy as np
from jax.experimental import pallas as pl
from jax.experimental.pallas import tpu as pltpu

_R = 512
_W = 3328
_W2 = 1664
_KNN = 32
_EPS = 1e-15
_BIG = 1e30


def _select_and_message(x_i, d, col, xf, mu3, is2, gvv, km):
    inf = jnp.float32(jnp.inf)

    def body(_, carry):
        d, acc = carry
        idx = jnp.argmin(d, axis=1).astype(jnp.int32).reshape(_R, 1)
        oh = col == idx
        x_j = jnp.dot(oh.astype(jnp.float32), xf,
                      preferred_element_type=jnp.float32)
        d = jnp.where(oh, inf, d)
        a = x_j - x_i
        gs = jnp.zeros((_R, 72), jnp.float32)
        for dd in range(3):
            diff = a[:, dd:dd + 1] - mu3[dd:dd + 1, :]
            gs = gs - diff * diff * is2[dd:dd + 1, :]
        w = jnp.exp(gs) * gvv
        w9 = jnp.dot(w, km, preferred_element_type=jnp.float32)
        msg = (x_j[:, 0:1] * w9[:, 0:3] + x_j[:, 1:2] * w9[:, 3:6]
               + x_j[:, 2:3] * w9[:, 6:9])
        return d, acc + msg

    _, acc = jax.lax.fori_loop(0, _KNN, body,
                               (d, jnp.zeros((_R, 3), jnp.float32)))
    return acc


def _epilogue(acc, x_i, rt_ref, bi_ref):
    return (acc * (1.0 / _KNN)
            + jnp.dot(x_i, rt_ref[...], preferred_element_type=jnp.float32)
            + bi_ref[...])


def _full_kernel(Np, xb_ref, xt_ref, xf_ref, bb_ref, bc_ref, mu_ref,
                 is2_ref, gv_ref, km_ref, rt_ref, bi_ref, out_ref):
    i = pl.program_id(0)
    x_i = xb_ref[...]
    xt = xt_ref[...]
    sq_i = jnp.sum(x_i * x_i, axis=1, keepdims=True)
    sq_c = jnp.sum(xt * xt, axis=0, keepdims=True)
    prod = jnp.dot(x_i, xt, preferred_element_type=jnp.float32)
    d = sq_i + sq_c - 2.0 * prod
    col = jax.lax.broadcasted_iota(jnp.int32, (_R, Np), 1)
    rowg = i * _R + jax.lax.broadcasted_iota(jnp.int32, (_R, Np), 0)
    valid = (bb_ref[...] == bc_ref[...]) & (col != rowg)
    d = jnp.where(valid, d, _BIG)
    acc = _select_and_message(x_i, d, col, xf_ref[...],
                              mu_ref[...], is2_ref[...], gv_ref[...],
                              km_ref[...])
    out_ref[...] = _epilogue(acc, x_i, rt_ref, bi_ref)


def _win_kernel(Np, W, s_ref, xb_ref, xt_ref, xf_ref, bb_ref, bc_ref, mu_ref,
                is2_ref, gv_ref, km_ref, rt_ref, bi_ref, out_ref):
    i = pl.program_id(0)
    s = pl.multiple_of(s_ref[i], 128)
    x_i = xb_ref[...]
    xt = xt_ref[:, pl.ds(s, W)]
    xf = xf_ref[pl.ds(s, W), :]
    bc = bc_ref[:, pl.ds(s, W)]
    sq_i = jnp.sum(x_i * x_i, axis=1, keepdims=True)
    sq_c = jnp.sum(xt * xt, axis=0, keepdims=True)
    prod = jnp.dot(x_i, xt, preferred_element_type=jnp.float32)
    d = sq_i + sq_c - 2.0 * prod
    lcol = jax.lax.broadcasted_iota(jnp.int32, (_R, W), 1)
    rowg = i * _R + jax.lax.broadcasted_iota(jnp.int32, (_R, W), 0)
    valid = (bb_ref[...] == bc) & (s + lcol != rowg)
    d = jnp.where(valid, d, _BIG)
    acc = _select_and_message(x_i, d, lcol, xf,
                              mu_ref[...], is2_ref[...], gv_ref[...],
                              km_ref[...])
    out_ref[...] = _epilogue(acc, x_i, rt_ref, bi_ref)


def _seg_kernel(Np, W, rs_ref, cs_ref, a_ref, b_ref, xt_ref, xf_ref, bp_ref,
                bc_ref, mu_ref, is2_ref, gv_ref, km_ref, rt_ref, bi_ref,
                out_ref):
    t = pl.program_id(0)
    a = a_ref[t]
    bnd = b_ref[t]

    @pl.when(bnd > a)
    def _():
        rs = pl.multiple_of(rs_ref[t], 8)
        s = pl.multiple_of(cs_ref[t], 128)
        x_i = xf_ref[pl.ds(rs, _R), :]
        bb = bp_ref[pl.ds(rs, _R), :]
        xt = xt_ref[:, pl.ds(s, W)]
        xf = xf_ref[pl.ds(s, W), :]
        bc = bc_ref[:, pl.ds(s, W)]
        sq_i = jnp.sum(x_i * x_i, axis=1, keepdims=True)
        sq_c = jnp.sum(xt * xt, axis=0, keepdims=True)
        prod = jnp.dot(x_i, xt, preferred_element_type=jnp.float32)
        d = sq_i + sq_c - 2.0 * prod
        lcol = jax.lax.broadcasted_iota(jnp.int32, (_R, W), 1)
        rowg = rs + jax.lax.broadcasted_iota(jnp.int32, (_R, W), 0)
        valid = (bb == bc) & (s + lcol != rowg)
        d = jnp.where(valid, d, _BIG)
        acc = _select_and_message(x_i, d, lcol, xf,
                                  mu_ref[...], is2_ref[...], gv_ref[...],
                                  km_ref[...])
        val = _epilogue(acc, x_i, rt_ref, bi_ref)
        ridx = rs + jax.lax.broadcasted_iota(jnp.int32, (_R, 1), 0)
        rmask = (ridx >= a) & (ridx < bnd)
        cur = out_ref[pl.ds(rs, _R), :]
        out_ref[pl.ds(rs, _R), :] = jnp.where(rmask, val, cur)


def _loss_kernel(o_ref, y_ref, out_ref):
    diff = o_ref[...] - y_ref[...]
    out_ref[...] = jnp.sum(diff * diff, keepdims=True).reshape(1, 1)


@jax.jit
def kernel(x, y, batch, g, mu, sigma, root_w, bias):
    n, f = x.shape
    np_pad = ((n + _R - 1) // _R) * _R
    nblk = np_pad // _R
    x_pad = jnp.pad(x, ((0, np_pad - n), (0, 0)))
    b32 = batch.astype(jnp.int32)
    b_pad = jnp.pad(b32, (0, np_pad - n), constant_values=-1)
    xt = x_pad.T
    b_rows = b_pad.reshape(np_pad, 1)
    b_cols = b_pad.reshape(1, np_pad)
    f_in, m_dim, k_dim, d_dim = mu.shape
    t = f_in * m_dim * k_dim
    mu3 = mu.reshape(t, d_dim).T
    is2 = (0.5 / (_EPS + sigma * sigma)).reshape(t, d_dim).T
    gv = g.reshape(1, t)
    km = jnp.asarray(np.repeat(np.eye(f_in * m_dim, dtype=np.float32),
                               k_dim, axis=0))
    rt = root_w.T
    bi = bias.reshape(1, f)

    first_b = b_pad[::_R]
    last_b = b32[jnp.minimum(jnp.arange(nblk) * _R + _R - 1, n - 1)]
    weff = min(_W, np_pad)
    starts = jnp.searchsorted(b32, first_b, side="left").astype(jnp.int32)
    ends = jnp.searchsorted(b32, last_b, side="right").astype(jnp.int32)
    starts = jnp.minimum((starts // 128) * 128, np_pad - weff)
    fits = jnp.max(ends - starts) <= weff
    vals = jnp.arange(8, dtype=jnp.int32)
    cnt = (jnp.searchsorted(b32, vals, side="right")
           - jnp.searchsorted(b32, vals, side="left"))
    seg_ok = jnp.min(jnp.where(cnt > 0, cnt, 33)) >= 33
    use_win = fits & seg_ok

    w2 = min(_W2, np_pad)
    seg_a = jnp.searchsorted(b32, vals, side="left").astype(jnp.int32)
    seg_b = jnp.searchsorted(b32, vals, side="right").astype(jnp.int32)
    nb8 = (seg_b - seg_a + 7 + _R - 1) // _R
    cum = jnp.cumsum(nb8)
    nbt = nblk + 9
    ts = jnp.arange(nbt, dtype=jnp.int32)
    segix = jnp.clip(jnp.searchsorted(cum, ts, side="right"), 0, 7)
    jj = ts - (cum[segix] - nb8[segix])
    a_t = seg_a[segix]
    b_t = seg_b[segix]
    rs_t = jnp.clip((a_t // 8) * 8 + _R * jj, 0, np_pad - _R).astype(jnp.int32)
    cs8 = jnp.clip((seg_a // 128) * 128, 0, np_pad - w2)
    cs_t = cs8[segix].astype(jnp.int32)
    valid_t = ts < cum[7]
    a_t = jnp.where(valid_t, a_t, 0).astype(jnp.int32)
    b_t = jnp.where(valid_t, b_t, 0).astype(jnp.int32)
    fits2 = jnp.max(jnp.where(cnt > 0, seg_b - cs8, 0)) <= w2
    use_seg = fits2 & seg_ok

    def make_ins(blocked):
        if blocked:
            blk = lambda: (lambda i: (i, 0))
            whole = lambda: (lambda i: (0, 0))
        else:
            blk = lambda: (lambda i, s: (i, 0))
            whole = lambda: (lambda i, s: (0, 0))
        return [
            pl.BlockSpec((_R, f), blk()),
            pl.BlockSpec((f, np_pad), whole()),
            pl.BlockSpec((np_pad, f), whole()),
            pl.BlockSpec((_R, 1), blk()),
            pl.BlockSpec((1, np_pad), whole()),
            pl.BlockSpec((d_dim, t), whole()),
            pl.BlockSpec((d_dim, t), whole()),
            pl.BlockSpec((1, t), whole()),
            pl.BlockSpec((t, f_in * m_dim), whole()),
            pl.BlockSpec((f, f), whole()),
            pl.BlockSpec((1, f), whole()),
        ]

    dense_args = (x_pad, xt, x_pad, b_rows, b_cols, mu3, is2, gv, km, rt, bi)
    out_sds = jax.ShapeDtypeStruct((np_pad, f), jnp.float32)

    def run_win(_):
        return pl.pallas_call(
            functools.partial(_win_kernel, np_pad, weff),
            grid_spec=pltpu.PrefetchScalarGridSpec(
                num_scalar_prefetch=1,
                grid=(nblk,),
                in_specs=make_ins(False),
                out_specs=pl.BlockSpec((_R, f), lambda i, s: (i, 0)),
            ),
            out_shape=out_sds,
        )(starts, *dense_args)

    def run_full(_):
        return pl.pallas_call(
            functools.partial(_full_kernel, np_pad),
            grid=(nblk,),
            in_specs=make_ins(True),
            out_specs=pl.BlockSpec((_R, f), lambda i: (i, 0)),
            out_shape=out_sds,
        )(*dense_args)

    def run_seg(_):
        whole = lambda: (lambda i, *_: (0, 0))
        seg_ins = [
            pl.BlockSpec((f, np_pad), whole()),
            pl.BlockSpec((np_pad, f), whole()),
            pl.BlockSpec((np_pad, 1), whole()),
            pl.BlockSpec((1, np_pad), whole()),
            pl.BlockSpec((d_dim, t), whole()),
            pl.BlockSpec((d_dim, t), whole()),
            pl.BlockSpec((1, t), whole()),
            pl.BlockSpec((t, f_in * m_dim), whole()),
            pl.BlockSpec((f, f), whole()),
            pl.BlockSpec((1, f), whole()),
        ]
        return pl.pallas_call(
            functools.partial(_seg_kernel, np_pad, w2),
            grid_spec=pltpu.PrefetchScalarGridSpec(
                num_scalar_prefetch=4,
                grid=(nbt,),
                in_specs=seg_ins,
                out_specs=pl.BlockSpec((np_pad, f), lambda i, *_: (0, 0)),
            ),
            out_shape=out_sds,
        )(rs_t, cs_t, a_t, b_t, xt, x_pad, b_rows, b_cols, mu3, is2, gv,
          km, rt, bi)

    out = jax.lax.cond(
        use_seg, run_seg,
        lambda _: jax.lax.cond(use_win, run_win, run_full, operand=None),
        operand=None)

    out = out[:n]
    sq = pl.pallas_call(
        _loss_kernel,
        out_shape=jax.ShapeDtypeStruct((1, 1), jnp.float32),
    )(out, y)
    loss = sq[0, 0] / (n * f)
    return out, loss
